# baseline (speedup 1.0000x reference)
import functools

import jax
import jax.numpy as jnp
import numpy as np
from jax import lax
from jax.experimental import pallas as pl
from jax.experimental.pallas import tpu as pltpu
from jax.experimental.pallas import tpu_sc as plsc

N_NODES = 50000
N_EDGES = 1600000
FEAT = 16

NUM_TILES = 32
CHUNK = 128
KBUF = 4
CPS = 80
SLABS = 5
ROWS_PER_TILE = CPS * SLABS
PAIRS = CPS // (2 * KBUF)
E_PAD = NUM_TILES * ROWS_PER_TILE * CHUNK
PAD_NODE = N_NODES

ACC_ROWS_PER_TILE = 3200
N_PAD = 16 * ACC_ROWS_PER_TILE
WB = 640
NPK = N_PAD // 8
VALID_PACK_ROWS = N_NODES // 8


def _sc_agg_body(table_hbm, src_hbm, dst_hbm, out_hbm,
                 acc, sidx, didx, rows_a, rows_b, zbuf,
                 sem_ga, sem_gb, sem_sa, sem_sb):
    c = lax.axis_index("c")
    s = lax.axis_index("s")
    tile = c * 16 + s

    def _zero_row(i, _):
        zbuf[i, :] = jnp.zeros((FEAT,), jnp.float32)
        return 0
    lax.fori_loop(0, WB, _zero_row, 0)
    accbase = s * ACC_ROWS_PER_TILE
    for q in range(ACC_ROWS_PER_TILE // WB):
        pltpu.sync_copy(zbuf, acc.at[pl.ds(accbase + q * WB, WB)])
    plsc.subcore_barrier()

    rowbase = tile * ROWS_PER_TILE

    def _fire_g(k, buf, j, sem):
        return pltpu.async_copy(table_hbm.at[sidx.at[k]], buf.at[j], sem)

    def _drain_g(k, buf, j, sem):
        pltpu.make_async_copy(table_hbm.at[sidx.at[k]], buf.at[j], sem).wait()

    def _fire_s(k, buf, j, sem):
        return pltpu.async_copy(buf.at[j], acc.at[didx.at[k]], sem, add=True)

    def _drain_s(k, buf, j, sem):
        pltpu.make_async_copy(buf.at[j], acc.at[didx.at[k]], sem).wait()

    def _slab(sl, _):
        base = rowbase + sl * CPS
        pltpu.sync_copy(src_hbm.at[pl.ds(base, CPS)], sidx)
        pltpu.sync_copy(dst_hbm.at[pl.ds(base, CPS)], didx)

        def _pair(p, _):
            ka = 2 * p * KBUF
            kb = ka + KBUF
            for j in range(KBUF):
                _fire_g(ka + j, rows_a, j, sem_ga)
            for j in range(KBUF):
                _fire_g(kb + j, rows_b, j, sem_gb)
            for j in range(KBUF):
                _drain_g(ka + j, rows_a, j, sem_ga)
            for j in range(KBUF):
                pltpu.sync_copy(rows_a.at[j], acc.at[didx.at[ka + j]],
                                add=True)
            for j in range(KBUF):
                _drain_g(kb + j, rows_b, j, sem_gb)
            for j in range(KBUF):
                pltpu.sync_copy(rows_b.at[j], acc.at[didx.at[kb + j]],
                                add=True)
            return 0

        lax.fori_loop(0, PAIRS, _pair, 0)
        return 0

    lax.fori_loop(0, SLABS, _slab, 0)
    plsc.subcore_barrier()

    pltpu.sync_copy(acc.at[pl.ds(accbase, ACC_ROWS_PER_TILE)],
                    out_hbm.at[c, pl.ds(accbase, ACC_ROWS_PER_TILE)])


@functools.cache
def _sc_aggregate():
    return pl.kernel(
        _sc_agg_body,
        out_type=jax.ShapeDtypeStruct((2, N_PAD, FEAT), jnp.float32),
        mesh=plsc.VectorSubcoreMesh(core_axis_name="c", subcore_axis_name="s"),
        compiler_params=pltpu.CompilerParams(use_tc_tiling_on_sc=False),
        scratch_types=[
            pltpu.VMEM_SHARED((N_PAD, FEAT), jnp.float32),
            pltpu.VMEM((CPS, CHUNK), jnp.int32),
            pltpu.VMEM((CPS, CHUNK), jnp.int32),
            pltpu.VMEM((KBUF, CHUNK, FEAT), jnp.float32),
            pltpu.VMEM((KBUF, CHUNK, FEAT), jnp.float32),
            pltpu.VMEM((WB, FEAT), jnp.float32),
            pltpu.SemaphoreType.DMA,
            pltpu.SemaphoreType.DMA,
            pltpu.SemaphoreType.DMA,
            pltpu.SemaphoreType.DMA,
        ],
    )


def _rowmask():
    r = lax.broadcasted_iota(jnp.int32, (NPK, 128), 0)
    return (r < VALID_PACK_ROWS).astype(jnp.float32)


def _bn(z, g_rep, gpack, bpack):
    m = jnp.dot(jnp.sum(z, axis=0, keepdims=True), g_rep,
                precision=lax.Precision.HIGHEST) * (1.0 / N_NODES)
    e2 = jnp.dot(jnp.sum(z * z, axis=0, keepdims=True), g_rep,
                 precision=lax.Precision.HIGHEST) * (1.0 / N_NODES)
    var = e2 - m * m
    return (z - m) * lax.rsqrt(var + 1e-5) * gpack + bpack


def _tc_layer1_body(a0, a1, hp, wl, bt, wr, sel, g_rep, gp, bp, out_h, out_cnt):
    agg = a0[...] + a1[...]
    cnt = jnp.maximum(
        jnp.dot(agg, sel[...], preferred_element_type=jnp.float32,
                precision=lax.Precision.HIGHEST), 1.0)
    mean = agg / cnt
    z = (jnp.dot(mean, wl[...], preferred_element_type=jnp.float32) + bt[...]
         + jnp.dot(hp[...], wr[...], preferred_element_type=jnp.float32))
    z = jnp.maximum(z, 0.0) * _rowmask()
    z = _bn(z, g_rep[...], gp[...], bp[...]) * _rowmask()
    out_h[...] = z
    out_cnt[...] = cnt


def _tc_mid_body(a0, a1, hp, cnt, wl, bt, wr, g_rep, gp, bp, out_h):
    mean = (a0[...] + a1[...]) / cnt[...]
    z = (jnp.dot(mean, wl[...], preferred_element_type=jnp.float32) + bt[...]
         + jnp.dot(hp[...], wr[...], preferred_element_type=jnp.float32))
    z = jnp.maximum(z, 0.0) * _rowmask()
    z = _bn(z, g_rep[...], gp[...], bp[...]) * _rowmask()
    out_h[...] = z


def _tc_last_body(a0, a1, hp, cnt, wl, bt, wr, out_h):
    mean = (a0[...] + a1[...]) / cnt[...]
    z = (jnp.dot(mean, wl[...], preferred_element_type=jnp.float32) + bt[...]
         + jnp.dot(hp[...], wr[...], preferred_element_type=jnp.float32))
    out_h[...] = jnp.maximum(z, 0.0)


_f32 = functools.partial(jax.ShapeDtypeStruct, dtype=jnp.float32)

_tc_layer1 = pl.pallas_call(
    _tc_layer1_body,
    out_shape=(_f32((NPK, 128)), _f32((NPK, 128))),
)
_tc_mid = pl.pallas_call(_tc_mid_body, out_shape=_f32((NPK, 128)))
_tc_last = pl.pallas_call(_tc_last_body, out_shape=_f32((NPK, 128)))

_G_REP = jnp.asarray(np.tile(np.eye(16, dtype=np.float32), (8, 8)))
_sel_np = np.zeros((128, 128), np.float32)
for _g in range(8):
    _sel_np[_g * 16 + 4, _g * 16:(_g + 1) * 16] = 1.0
_SEL = jnp.asarray(_sel_np)


def _bd(w16):
    return jnp.kron(jnp.eye(8, dtype=jnp.float32), w16)


def _tile128(v16):
    return jnp.tile(v16, 8)[None, :]


def kernel(x, edge_index, W1l, b1, W1r, Wl, bl, Wr, gamma, beta):
    src = edge_index[0]
    dst = edge_index[1]
    pad = E_PAD - N_EDGES
    srcp = jnp.concatenate(
        [src, jnp.full((pad,), PAD_NODE, jnp.int32)]).reshape(-1, CHUNK)
    dstp = jnp.concatenate(
        [dst, jnp.full((pad,), PAD_NODE, jnp.int32)]).reshape(-1, CHUNK)

    t1 = jnp.zeros((N_PAD, FEAT), jnp.float32)
    t1 = t1.at[:N_NODES, :4].set(x)
    t1 = t1.at[:N_NODES, 4].set(1.0)

    w1l_pad = jnp.zeros((16, 16), jnp.float32).at[:, :4].set(W1l)
    w1r_pad = jnp.zeros((16, 16), jnp.float32).at[:, :4].set(W1r)

    agg = _sc_aggregate()(t1, srcp, dstp)
    h, cnt = _tc_layer1(
        agg[0].reshape(NPK, 128), agg[1].reshape(NPK, 128),
        t1.reshape(NPK, 128), _bd(w1l_pad.T), _tile128(b1), _bd(w1r_pad.T),
        _SEL, _G_REP, _tile128(gamma[0]), _tile128(beta[0]))

    for i in range(6):
        agg = _sc_aggregate()(h.reshape(N_PAD, FEAT), srcp, dstp)
        h = _tc_mid(
            agg[0].reshape(NPK, 128), agg[1].reshape(NPK, 128), h, cnt,
            _bd(Wl[i].T), _tile128(bl[i]), _bd(Wr[i].T),
            _G_REP, _tile128(gamma[i + 1]), _tile128(beta[i + 1]))

    agg = _sc_aggregate()(h.reshape(N_PAD, FEAT), srcp, dstp)
    h = _tc_last(
        agg[0].reshape(NPK, 128), agg[1].reshape(NPK, 128), h, cnt,
        _bd(Wl[6].T), _tile128(bl[6]), _bd(Wr[6].T))

    return h.reshape(N_PAD, FEAT)[:N_NODES]


# speedup vs baseline: 1.3143x; 1.3143x over previous
import functools

import jax
import jax.numpy as jnp
import numpy as np
from jax import lax
from jax.experimental import pallas as pl
from jax.experimental.pallas import tpu as pltpu
from jax.experimental.pallas import tpu_sc as plsc

N_NODES = 50000
N_EDGES = 1600000
FEAT = 16

NUM_TILES = 32
CHUNK = 128
KBUF = 8
ROWS_PER_TILE = 392
NGROUPS = ROWS_PER_TILE // KBUF
E_PAD = NUM_TILES * ROWS_PER_TILE * CHUNK
PAD_NODE = N_NODES

ACC_ROWS_PER_TILE = 3200
N_PAD = 16 * ACC_ROWS_PER_TILE
WB = 640
NPK = N_PAD // 8
VALID_PACK_ROWS = N_NODES // 8


def _sc_agg_body(table_hbm, src_hbm, dst_hbm, out_hbm,
                 acc, sidx, didx, rows_a, rows_b, zbuf,
                 sem_ga, sem_gb, sem_sa, sem_sb):
    c = lax.axis_index("c")
    s = lax.axis_index("s")
    tile = c * 16 + s

    def _zero_row(i, _):
        zbuf[i, :] = jnp.zeros((FEAT,), jnp.float32)
        return 0
    lax.fori_loop(0, WB, _zero_row, 0)
    accbase = s * ACC_ROWS_PER_TILE
    for q in range(ACC_ROWS_PER_TILE // WB):
        pltpu.sync_copy(zbuf, acc.at[pl.ds(accbase + q * WB, WB)])
    plsc.subcore_barrier()

    rowbase = tile * ROWS_PER_TILE

    def _group(g, _):
        r0 = rowbase + g * KBUF
        pltpu.sync_copy(src_hbm.at[pl.ds(r0, KBUF)], sidx)
        pltpu.sync_copy(dst_hbm.at[pl.ds(r0, KBUF)], didx)
        copies = [pltpu.async_copy(table_hbm.at[sidx.at[j]], rows_a.at[j],
                                   sem_ga) for j in range(KBUF)]
        for cp in copies:
            cp.wait()
        for j in range(KBUF):
            pltpu.sync_copy(rows_a.at[j], acc.at[didx.at[j]], add=True)
        return 0

    lax.fori_loop(0, NGROUPS, _group, 0)
    plsc.subcore_barrier()

    pltpu.sync_copy(acc.at[pl.ds(accbase, ACC_ROWS_PER_TILE)],
                    out_hbm.at[c, pl.ds(accbase, ACC_ROWS_PER_TILE)])


@functools.cache
def _sc_aggregate():
    return pl.kernel(
        _sc_agg_body,
        out_type=jax.ShapeDtypeStruct((2, N_PAD, FEAT), jnp.float32),
        mesh=plsc.VectorSubcoreMesh(core_axis_name="c", subcore_axis_name="s"),
        compiler_params=pltpu.CompilerParams(use_tc_tiling_on_sc=False),
        scratch_types=[
            pltpu.VMEM_SHARED((N_PAD, FEAT), jnp.float32),
            pltpu.VMEM((KBUF, CHUNK), jnp.int32),
            pltpu.VMEM((KBUF, CHUNK), jnp.int32),
            pltpu.VMEM((KBUF, CHUNK, FEAT), jnp.float32),
            pltpu.VMEM((KBUF, CHUNK, FEAT), jnp.float32),
            pltpu.VMEM((WB, FEAT), jnp.float32),
            pltpu.SemaphoreType.DMA,
            pltpu.SemaphoreType.DMA,
            pltpu.SemaphoreType.DMA,
            pltpu.SemaphoreType.DMA,
        ],
    )


def _rowmask():
    r = lax.broadcasted_iota(jnp.int32, (NPK, 128), 0)
    return (r < VALID_PACK_ROWS).astype(jnp.float32)


def _bn(z, g_rep, gpack, bpack):
    m = jnp.dot(jnp.sum(z, axis=0, keepdims=True), g_rep,
                precision=lax.Precision.HIGHEST) * (1.0 / N_NODES)
    e2 = jnp.dot(jnp.sum(z * z, axis=0, keepdims=True), g_rep,
                 precision=lax.Precision.HIGHEST) * (1.0 / N_NODES)
    var = e2 - m * m
    return (z - m) * lax.rsqrt(var + 1e-5) * gpack + bpack


def _tc_layer1_body(a0, a1, hp, wl, bt, wr, sel, g_rep, gp, bp, out_h, out_cnt):
    agg = a0[...] + a1[...]
    cnt = jnp.maximum(
        jnp.dot(agg, sel[...], preferred_element_type=jnp.float32,
                precision=lax.Precision.HIGHEST), 1.0)
    mean = agg / cnt
    z = (jnp.dot(mean, wl[...], preferred_element_type=jnp.float32) + bt[...]
         + jnp.dot(hp[...], wr[...], preferred_element_type=jnp.float32))
    z = jnp.maximum(z, 0.0) * _rowmask()
    z = _bn(z, g_rep[...], gp[...], bp[...]) * _rowmask()
    out_h[...] = z
    out_cnt[...] = cnt


def _tc_mid_body(a0, a1, hp, cnt, wl, bt, wr, g_rep, gp, bp, out_h):
    mean = (a0[...] + a1[...]) / cnt[...]
    z = (jnp.dot(mean, wl[...], preferred_element_type=jnp.float32) + bt[...]
         + jnp.dot(hp[...], wr[...], preferred_element_type=jnp.float32))
    z = jnp.maximum(z, 0.0) * _rowmask()
    z = _bn(z, g_rep[...], gp[...], bp[...]) * _rowmask()
    out_h[...] = z


def _tc_last_body(a0, a1, hp, cnt, wl, bt, wr, out_h):
    mean = (a0[...] + a1[...]) / cnt[...]
    z = (jnp.dot(mean, wl[...], preferred_element_type=jnp.float32) + bt[...]
         + jnp.dot(hp[...], wr[...], preferred_element_type=jnp.float32))
    out_h[...] = jnp.maximum(z, 0.0)


_f32 = functools.partial(jax.ShapeDtypeStruct, dtype=jnp.float32)

_tc_layer1 = pl.pallas_call(
    _tc_layer1_body,
    out_shape=(_f32((NPK, 128)), _f32((NPK, 128))),
)
_tc_mid = pl.pallas_call(_tc_mid_body, out_shape=_f32((NPK, 128)))
_tc_last = pl.pallas_call(_tc_last_body, out_shape=_f32((NPK, 128)))

_G_REP = jnp.asarray(np.tile(np.eye(16, dtype=np.float32), (8, 8)))
_sel_np = np.zeros((128, 128), np.float32)
for _g in range(8):
    _sel_np[_g * 16 + 4, _g * 16:(_g + 1) * 16] = 1.0
_SEL = jnp.asarray(_sel_np)


def _bd(w16):
    return jnp.kron(jnp.eye(8, dtype=jnp.float32), w16)


def _tile128(v16):
    return jnp.tile(v16, 8)[None, :]


def kernel(x, edge_index, W1l, b1, W1r, Wl, bl, Wr, gamma, beta):
    src = edge_index[0]
    dst = edge_index[1]
    pad = E_PAD - N_EDGES
    srcp = jnp.concatenate(
        [src, jnp.full((pad,), PAD_NODE, jnp.int32)]).reshape(-1, CHUNK)
    dstp = jnp.concatenate(
        [dst, jnp.full((pad,), PAD_NODE, jnp.int32)]).reshape(-1, CHUNK)

    t1 = jnp.zeros((N_PAD, FEAT), jnp.float32)
    t1 = t1.at[:N_NODES, :4].set(x)
    t1 = t1.at[:N_NODES, 4].set(1.0)

    w1l_pad = jnp.zeros((16, 16), jnp.float32).at[:, :4].set(W1l)
    w1r_pad = jnp.zeros((16, 16), jnp.float32).at[:, :4].set(W1r)

    agg = _sc_aggregate()(t1, srcp, dstp)
    h, cnt = _tc_layer1(
        agg[0].reshape(NPK, 128), agg[1].reshape(NPK, 128),
        t1.reshape(NPK, 128), _bd(w1l_pad.T), _tile128(b1), _bd(w1r_pad.T),
        _SEL, _G_REP, _tile128(gamma[0]), _tile128(beta[0]))

    for i in range(6):
        agg = _sc_aggregate()(h.reshape(N_PAD, FEAT), srcp, dstp)
        h = _tc_mid(
            agg[0].reshape(NPK, 128), agg[1].reshape(NPK, 128), h, cnt,
            _bd(Wl[i].T), _tile128(bl[i]), _bd(Wr[i].T),
            _G_REP, _tile128(gamma[i + 1]), _tile128(beta[i + 1]))

    agg = _sc_aggregate()(h.reshape(N_PAD, FEAT), srcp, dstp)
    h = _tc_last(
        agg[0].reshape(NPK, 128), agg[1].reshape(NPK, 128), h, cnt,
        _bd(Wl[6].T), _tile128(bl[6]), _bd(Wr[6].T))

    return h.reshape(N_PAD, FEAT)[:N_NODES]


# speedup vs baseline: 1.4568x; 1.1085x over previous
import functools

import jax
import jax.numpy as jnp
import numpy as np
from jax import lax
from jax.experimental import pallas as pl
from jax.experimental.pallas import tpu as pltpu
from jax.experimental.pallas import tpu_sc as plsc

N_NODES = 50000
N_EDGES = 1600000
FEAT = 16

NUM_TILES = 32
CHUNK = 128
KBUF = 4
ROWS_PER_TILE = 392
PAIRS = ROWS_PER_TILE // (2 * KBUF)
E_PAD = NUM_TILES * ROWS_PER_TILE * CHUNK
PAD_NODE = N_NODES

ACC_ROWS_PER_TILE = 3200
N_PAD = 16 * ACC_ROWS_PER_TILE
WB = 640
NPK = N_PAD // 8
VALID_PACK_ROWS = N_NODES // 8


def _sc_agg_body(table_hbm, src_hbm, dst_hbm, out_hbm,
                 acc, sidx_a, didx_a, sidx_b, didx_b, rows_a, rows_b, zbuf,
                 sem_ga, sem_gb, sem_sa, sem_sb):
    c = lax.axis_index("c")
    s = lax.axis_index("s")
    tile = c * 16 + s

    def _zero_row(i, _):
        zbuf[i, :] = jnp.zeros((FEAT,), jnp.float32)
        return 0
    lax.fori_loop(0, WB, _zero_row, 0)
    accbase = s * ACC_ROWS_PER_TILE
    for q in range(ACC_ROWS_PER_TILE // WB):
        pltpu.sync_copy(zbuf, acc.at[pl.ds(accbase + q * WB, WB)])
    plsc.subcore_barrier()

    rowbase = tile * ROWS_PER_TILE

    def _copy_idx(r0, sb, db):
        pltpu.sync_copy(src_hbm.at[pl.ds(r0, KBUF)], sb)
        pltpu.sync_copy(dst_hbm.at[pl.ds(r0, KBUF)], db)

    def _fire_g(sb, buf, sem):
        return [pltpu.async_copy(table_hbm.at[sb.at[j]], buf.at[j], sem)
                for j in range(KBUF)]

    def _drain_g(sb, buf, sem):
        for j in range(KBUF):
            pltpu.make_async_copy(table_hbm.at[sb.at[j]], buf.at[j],
                                  sem).wait()

    def _fire_s(db, buf, sem):
        return [pltpu.async_copy(buf.at[j], acc.at[db.at[j]], sem, add=True)
                for j in range(KBUF)]

    def _drain_s(db, buf, sem):
        for j in range(KBUF):
            pltpu.make_async_copy(buf.at[j], acc.at[db.at[j]], sem).wait()

    _copy_idx(rowbase, sidx_a, didx_a)
    _fire_g(sidx_a, rows_a, sem_ga)

    def _pair(p, _):
        r0 = rowbase + 2 * p * KBUF

        @pl.when(p > 0)
        def _():
            _drain_s(didx_b, rows_b, sem_sb)
        _copy_idx(r0 + KBUF, sidx_b, didx_b)
        _fire_g(sidx_b, rows_b, sem_gb)
        _drain_g(sidx_a, rows_a, sem_ga)
        _fire_s(didx_a, rows_a, sem_sa)
        _drain_g(sidx_b, rows_b, sem_gb)
        _fire_s(didx_b, rows_b, sem_sb)

        @pl.when(p < PAIRS - 1)
        def _():
            _drain_s(didx_a, rows_a, sem_sa)
            _copy_idx(r0 + 2 * KBUF, sidx_a, didx_a)
            _fire_g(sidx_a, rows_a, sem_ga)
        return 0

    lax.fori_loop(0, PAIRS, _pair, 0)
    _drain_s(didx_a, rows_a, sem_sa)
    _drain_s(didx_b, rows_b, sem_sb)
    plsc.subcore_barrier()

    pltpu.sync_copy(acc.at[pl.ds(accbase, ACC_ROWS_PER_TILE)],
                    out_hbm.at[c, pl.ds(accbase, ACC_ROWS_PER_TILE)])


@functools.cache
def _sc_aggregate():
    return pl.kernel(
        _sc_agg_body,
        out_type=jax.ShapeDtypeStruct((2, N_PAD, FEAT), jnp.float32),
        mesh=plsc.VectorSubcoreMesh(core_axis_name="c", subcore_axis_name="s"),
        compiler_params=pltpu.CompilerParams(use_tc_tiling_on_sc=False),
        scratch_types=[
            pltpu.VMEM_SHARED((N_PAD, FEAT), jnp.float32),
            pltpu.VMEM((KBUF, CHUNK), jnp.int32),
            pltpu.VMEM((KBUF, CHUNK), jnp.int32),
            pltpu.VMEM((KBUF, CHUNK), jnp.int32),
            pltpu.VMEM((KBUF, CHUNK), jnp.int32),
            pltpu.VMEM((KBUF, CHUNK, FEAT), jnp.float32),
            pltpu.VMEM((KBUF, CHUNK, FEAT), jnp.float32),
            pltpu.VMEM((WB, FEAT), jnp.float32),
            pltpu.SemaphoreType.DMA,
            pltpu.SemaphoreType.DMA,
            pltpu.SemaphoreType.DMA,
            pltpu.SemaphoreType.DMA,
        ],
    )


def _rowmask():
    r = lax.broadcasted_iota(jnp.int32, (NPK, 128), 0)
    return (r < VALID_PACK_ROWS).astype(jnp.float32)


def _bn(z, g_rep, gpack, bpack):
    m = jnp.dot(jnp.sum(z, axis=0, keepdims=True), g_rep,
                precision=lax.Precision.HIGHEST) * (1.0 / N_NODES)
    e2 = jnp.dot(jnp.sum(z * z, axis=0, keepdims=True), g_rep,
                 precision=lax.Precision.HIGHEST) * (1.0 / N_NODES)
    var = e2 - m * m
    return (z - m) * lax.rsqrt(var + 1e-5) * gpack + bpack


def _tc_layer1_body(a0, a1, hp, wl, bt, wr, sel, g_rep, gp, bp, out_h, out_cnt):
    agg = a0[...] + a1[...]
    cnt = jnp.maximum(
        jnp.dot(agg, sel[...], preferred_element_type=jnp.float32,
                precision=lax.Precision.HIGHEST), 1.0)
    mean = agg / cnt
    z = (jnp.dot(mean, wl[...], preferred_element_type=jnp.float32) + bt[...]
         + jnp.dot(hp[...], wr[...], preferred_element_type=jnp.float32))
    z = jnp.maximum(z, 0.0) * _rowmask()
    z = _bn(z, g_rep[...], gp[...], bp[...]) * _rowmask()
    out_h[...] = z
    out_cnt[...] = cnt


def _tc_mid_body(a0, a1, hp, cnt, wl, bt, wr, g_rep, gp, bp, out_h):
    mean = (a0[...] + a1[...]) / cnt[...]
    z = (jnp.dot(mean, wl[...], preferred_element_type=jnp.float32) + bt[...]
         + jnp.dot(hp[...], wr[...], preferred_element_type=jnp.float32))
    z = jnp.maximum(z, 0.0) * _rowmask()
    z = _bn(z, g_rep[...], gp[...], bp[...]) * _rowmask()
    out_h[...] = z


def _tc_last_body(a0, a1, hp, cnt, wl, bt, wr, out_h):
    mean = (a0[...] + a1[...]) / cnt[...]
    z = (jnp.dot(mean, wl[...], preferred_element_type=jnp.float32) + bt[...]
         + jnp.dot(hp[...], wr[...], preferred_element_type=jnp.float32))
    out_h[...] = jnp.maximum(z, 0.0)


_f32 = functools.partial(jax.ShapeDtypeStruct, dtype=jnp.float32)

_tc_layer1 = pl.pallas_call(
    _tc_layer1_body,
    out_shape=(_f32((NPK, 128)), _f32((NPK, 128))),
)
_tc_mid = pl.pallas_call(_tc_mid_body, out_shape=_f32((NPK, 128)))
_tc_last = pl.pallas_call(_tc_last_body, out_shape=_f32((NPK, 128)))

_G_REP = jnp.asarray(np.tile(np.eye(16, dtype=np.float32), (8, 8)))
_sel_np = np.zeros((128, 128), np.float32)
for _g in range(8):
    _sel_np[_g * 16 + 4, _g * 16:(_g + 1) * 16] = 1.0
_SEL = jnp.asarray(_sel_np)


def _bd(w16):
    return jnp.kron(jnp.eye(8, dtype=jnp.float32), w16)


def _tile128(v16):
    return jnp.tile(v16, 8)[None, :]


def kernel(x, edge_index, W1l, b1, W1r, Wl, bl, Wr, gamma, beta):
    src = edge_index[0]
    dst = edge_index[1]
    pad = E_PAD - N_EDGES
    srcp = jnp.concatenate(
        [src, jnp.full((pad,), PAD_NODE, jnp.int32)]).reshape(-1, CHUNK)
    dstp = jnp.concatenate(
        [dst, jnp.full((pad,), PAD_NODE, jnp.int32)]).reshape(-1, CHUNK)

    t1 = jnp.zeros((N_PAD, FEAT), jnp.float32)
    t1 = t1.at[:N_NODES, :4].set(x)
    t1 = t1.at[:N_NODES, 4].set(1.0)

    w1l_pad = jnp.zeros((16, 16), jnp.float32).at[:, :4].set(W1l)
    w1r_pad = jnp.zeros((16, 16), jnp.float32).at[:, :4].set(W1r)

    agg = _sc_aggregate()(t1, srcp, dstp)
    h, cnt = _tc_layer1(
        agg[0].reshape(NPK, 128), agg[1].reshape(NPK, 128),
        t1.reshape(NPK, 128), _bd(w1l_pad.T), _tile128(b1), _bd(w1r_pad.T),
        _SEL, _G_REP, _tile128(gamma[0]), _tile128(beta[0]))

    for i in range(6):
        agg = _sc_aggregate()(h.reshape(N_PAD, FEAT), srcp, dstp)
        h = _tc_mid(
            agg[0].reshape(NPK, 128), agg[1].reshape(NPK, 128), h, cnt,
            _bd(Wl[i].T), _tile128(bl[i]), _bd(Wr[i].T),
            _G_REP, _tile128(gamma[i + 1]), _tile128(beta[i + 1]))

    agg = _sc_aggregate()(h.reshape(N_PAD, FEAT), srcp, dstp)
    h = _tc_last(
        agg[0].reshape(NPK, 128), agg[1].reshape(NPK, 128), h, cnt,
        _bd(Wl[6].T), _tile128(bl[6]), _bd(Wr[6].T))

    return h.reshape(N_PAD, FEAT)[:N_NODES]


# speedup vs baseline: 1.6767x; 1.1510x over previous
import functools

import jax
import jax.numpy as jnp
import numpy as np
from jax import lax
from jax.experimental import pallas as pl
from jax.experimental.pallas import tpu as pltpu
from jax.experimental.pallas import tpu_sc as plsc

N_NODES = 50000
N_EDGES = 1600000
FEAT = 16

NUM_TILES = 32
CHUNK = 128
KBUF = 7
ROWS_PER_TILE = 392
PAIRS = ROWS_PER_TILE // (2 * KBUF)
E_PAD = NUM_TILES * ROWS_PER_TILE * CHUNK
PAD_NODE = N_NODES

ACC_ROWS_PER_TILE = 3200
N_PAD = 16 * ACC_ROWS_PER_TILE
WB = 640
NPK = N_PAD // 8
VALID_PACK_ROWS = N_NODES // 8


def _sc_agg_body(table_hbm, src_hbm, dst_hbm, out_hbm,
                 acc, sidx_a, didx_a, sidx_b, didx_b, rows_a, rows_b, zbuf,
                 sem_ga, sem_gb, sem_sa, sem_sb):
    c = lax.axis_index("c")
    s = lax.axis_index("s")
    tile = c * 16 + s

    def _zero_row(i, _):
        zbuf[i, :] = jnp.zeros((FEAT,), jnp.float32)
        return 0
    lax.fori_loop(0, WB, _zero_row, 0)
    accbase = s * ACC_ROWS_PER_TILE
    for q in range(ACC_ROWS_PER_TILE // WB):
        pltpu.sync_copy(zbuf, acc.at[pl.ds(accbase + q * WB, WB)])
    plsc.subcore_barrier()

    rowbase = tile * ROWS_PER_TILE

    def _copy_idx(r0, sb, db):
        pltpu.sync_copy(src_hbm.at[pl.ds(r0, KBUF)], sb)
        pltpu.sync_copy(dst_hbm.at[pl.ds(r0, KBUF)], db)

    def _fire_g(sb, buf, sem):
        return [pltpu.async_copy(table_hbm.at[sb.at[j]], buf.at[j], sem)
                for j in range(KBUF)]

    def _drain_g(sb, buf, sem):
        for j in range(KBUF):
            pltpu.make_async_copy(table_hbm.at[sb.at[j]], buf.at[j],
                                  sem).wait()

    def _fire_s(db, buf, sem):
        return [pltpu.async_copy(buf.at[j], acc.at[db.at[j]], sem, add=True)
                for j in range(KBUF)]

    def _drain_s(db, buf, sem):
        for j in range(KBUF):
            pltpu.make_async_copy(buf.at[j], acc.at[db.at[j]], sem).wait()

    _copy_idx(rowbase, sidx_a, didx_a)
    _fire_g(sidx_a, rows_a, sem_ga)

    def _pair(p, _):
        r0 = rowbase + 2 * p * KBUF

        @pl.when(p > 0)
        def _():
            _drain_s(didx_b, rows_b, sem_sb)
        _copy_idx(r0 + KBUF, sidx_b, didx_b)
        _fire_g(sidx_b, rows_b, sem_gb)
        _drain_g(sidx_a, rows_a, sem_ga)
        _fire_s(didx_a, rows_a, sem_sa)
        _drain_g(sidx_b, rows_b, sem_gb)
        _fire_s(didx_b, rows_b, sem_sb)

        @pl.when(p < PAIRS - 1)
        def _():
            _drain_s(didx_a, rows_a, sem_sa)
            _copy_idx(r0 + 2 * KBUF, sidx_a, didx_a)
            _fire_g(sidx_a, rows_a, sem_ga)
        return 0

    lax.fori_loop(0, PAIRS, _pair, 0)
    _drain_s(didx_a, rows_a, sem_sa)
    _drain_s(didx_b, rows_b, sem_sb)
    plsc.subcore_barrier()

    pltpu.sync_copy(acc.at[pl.ds(accbase, ACC_ROWS_PER_TILE)],
                    out_hbm.at[c, pl.ds(accbase, ACC_ROWS_PER_TILE)])


@functools.cache
def _sc_aggregate():
    return pl.kernel(
        _sc_agg_body,
        out_type=jax.ShapeDtypeStruct((2, N_PAD, FEAT), jnp.float32),
        mesh=plsc.VectorSubcoreMesh(core_axis_name="c", subcore_axis_name="s"),
        compiler_params=pltpu.CompilerParams(use_tc_tiling_on_sc=False),
        scratch_types=[
            pltpu.VMEM_SHARED((N_PAD, FEAT), jnp.float32),
            pltpu.VMEM((KBUF, CHUNK), jnp.int32),
            pltpu.VMEM((KBUF, CHUNK), jnp.int32),
            pltpu.VMEM((KBUF, CHUNK), jnp.int32),
            pltpu.VMEM((KBUF, CHUNK), jnp.int32),
            pltpu.VMEM((KBUF, CHUNK, FEAT), jnp.float32),
            pltpu.VMEM((KBUF, CHUNK, FEAT), jnp.float32),
            pltpu.VMEM((WB, FEAT), jnp.float32),
            pltpu.SemaphoreType.DMA,
            pltpu.SemaphoreType.DMA,
            pltpu.SemaphoreType.DMA,
            pltpu.SemaphoreType.DMA,
        ],
    )


def _rowmask():
    r = lax.broadcasted_iota(jnp.int32, (NPK, 128), 0)
    return (r < VALID_PACK_ROWS).astype(jnp.float32)


def _bn(z, g_rep, gpack, bpack):
    m = jnp.dot(jnp.sum(z, axis=0, keepdims=True), g_rep,
                precision=lax.Precision.HIGHEST) * (1.0 / N_NODES)
    e2 = jnp.dot(jnp.sum(z * z, axis=0, keepdims=True), g_rep,
                 precision=lax.Precision.HIGHEST) * (1.0 / N_NODES)
    var = e2 - m * m
    return (z - m) * lax.rsqrt(var + 1e-5) * gpack + bpack


def _tc_layer1_body(a0, a1, hp, wl, bt, wr, sel, g_rep, gp, bp, out_h, out_cnt):
    agg = a0[...] + a1[...]
    cnt = jnp.maximum(
        jnp.dot(agg, sel[...], preferred_element_type=jnp.float32,
                precision=lax.Precision.HIGHEST), 1.0)
    mean = agg / cnt
    z = (jnp.dot(mean, wl[...], preferred_element_type=jnp.float32) + bt[...]
         + jnp.dot(hp[...], wr[...], preferred_element_type=jnp.float32))
    z = jnp.maximum(z, 0.0) * _rowmask()
    z = _bn(z, g_rep[...], gp[...], bp[...]) * _rowmask()
    out_h[...] = z
    out_cnt[...] = cnt


def _tc_mid_body(a0, a1, hp, cnt, wl, bt, wr, g_rep, gp, bp, out_h):
    mean = (a0[...] + a1[...]) / cnt[...]
    z = (jnp.dot(mean, wl[...], preferred_element_type=jnp.float32) + bt[...]
         + jnp.dot(hp[...], wr[...], preferred_element_type=jnp.float32))
    z = jnp.maximum(z, 0.0) * _rowmask()
    z = _bn(z, g_rep[...], gp[...], bp[...]) * _rowmask()
    out_h[...] = z


def _tc_last_body(a0, a1, hp, cnt, wl, bt, wr, out_h):
    mean = (a0[...] + a1[...]) / cnt[...]
    z = (jnp.dot(mean, wl[...], preferred_element_type=jnp.float32) + bt[...]
         + jnp.dot(hp[...], wr[...], preferred_element_type=jnp.float32))
    out_h[...] = jnp.maximum(z, 0.0)


_f32 = functools.partial(jax.ShapeDtypeStruct, dtype=jnp.float32)

_tc_layer1 = pl.pallas_call(
    _tc_layer1_body,
    out_shape=(_f32((NPK, 128)), _f32((NPK, 128))),
)
_tc_mid = pl.pallas_call(_tc_mid_body, out_shape=_f32((NPK, 128)))
_tc_last = pl.pallas_call(_tc_last_body, out_shape=_f32((NPK, 128)))

_G_REP = jnp.asarray(np.tile(np.eye(16, dtype=np.float32), (8, 8)))
_sel_np = np.zeros((128, 128), np.float32)
for _g in range(8):
    _sel_np[_g * 16 + 4, _g * 16:(_g + 1) * 16] = 1.0
_SEL = jnp.asarray(_sel_np)


def _bd(w16):
    return jnp.kron(jnp.eye(8, dtype=jnp.float32), w16)


def _tile128(v16):
    return jnp.tile(v16, 8)[None, :]


def kernel(x, edge_index, W1l, b1, W1r, Wl, bl, Wr, gamma, beta):
    src = edge_index[0]
    dst = edge_index[1]
    pad = E_PAD - N_EDGES
    srcp = jnp.concatenate(
        [src, jnp.full((pad,), PAD_NODE, jnp.int32)]).reshape(-1, CHUNK)
    dstp = jnp.concatenate(
        [dst, jnp.full((pad,), PAD_NODE, jnp.int32)]).reshape(-1, CHUNK)

    t1 = jnp.zeros((N_PAD, FEAT), jnp.float32)
    t1 = t1.at[:N_NODES, :4].set(x)
    t1 = t1.at[:N_NODES, 4].set(1.0)

    w1l_pad = jnp.zeros((16, 16), jnp.float32).at[:, :4].set(W1l)
    w1r_pad = jnp.zeros((16, 16), jnp.float32).at[:, :4].set(W1r)

    agg = _sc_aggregate()(t1, srcp, dstp)
    h, cnt = _tc_layer1(
        agg[0].reshape(NPK, 128), agg[1].reshape(NPK, 128),
        t1.reshape(NPK, 128), _bd(w1l_pad.T), _tile128(b1), _bd(w1r_pad.T),
        _SEL, _G_REP, _tile128(gamma[0]), _tile128(beta[0]))

    for i in range(6):
        agg = _sc_aggregate()(h.reshape(N_PAD, FEAT), srcp, dstp)
        h = _tc_mid(
            agg[0].reshape(NPK, 128), agg[1].reshape(NPK, 128), h, cnt,
            _bd(Wl[i].T), _tile128(bl[i]), _bd(Wr[i].T),
            _G_REP, _tile128(gamma[i + 1]), _tile128(beta[i + 1]))

    agg = _sc_aggregate()(h.reshape(N_PAD, FEAT), srcp, dstp)
    h = _tc_last(
        agg[0].reshape(NPK, 128), agg[1].reshape(NPK, 128), h, cnt,
        _bd(Wl[6].T), _tile128(bl[6]), _bd(Wr[6].T))

    return h.reshape(N_PAD, FEAT)[:N_NODES]
